# Initial kernel scaffold; baseline (speedup 1.0000x reference)
import jax
import jax.numpy as jnp
from jax.experimental import pallas as pl


def kernel(node_ids, edge_index1, etype1, norm1, edge_index2, etype2, norm2, emb, basis1, coeff1, bias1, basis2, coeff2, bias2):
    raise NotImplementedError("write your pallas kernel here")


# speedup vs baseline: 9.6635x; 9.6635x over previous
import functools

import jax
import jax.numpy as jnp
from jax import lax
from jax.experimental import pallas as pl
from jax.experimental.pallas import tpu as pltpu
from jax.experimental.pallas import tpu_sc as plsc

H = 128
NPAD = 10240
NC = 2
NS = 16
L = 16
NW = NC * NS
CHUNK = 128
BN = 512
SB = 16
SLAB = 96
HOLE_LO, HOLE_HI = 56, 72


def _wsum_body(coeff_ref, basis_ref, out_ref):
    out_ref[...] = jnp.dot(coeff_ref[...], basis_ref[...],
                           preferred_element_type=jnp.float32,
                           precision=lax.Precision.HIGHEST)


def _basis_weights(coeff, basis):
    r = coeff.shape[0]
    bf = basis.reshape(r, -1)
    out = pl.pallas_call(
        _wsum_body,
        out_shape=jax.ShapeDtypeStruct((r, bf.shape[1]), jnp.float32),
    )(coeff, bf)
    return out.reshape(basis.shape)


def _xform1_body(nrel, x_ref, w_ref, out_ref):
    a = x_ref[...]
    for r in range(nrel):
        out_ref[r] = jnp.dot(a, w_ref[r], preferred_element_type=jnp.float32,
                             precision=lax.Precision.HIGHEST)


def _transform1(x, w):
    nrel = w.shape[0]
    grid = NPAD // BN
    return pl.pallas_call(
        functools.partial(_xform1_body, nrel),
        grid=(grid,),
        in_specs=[
            pl.BlockSpec((BN, H), lambda i: (i, 0)),
            pl.BlockSpec((nrel, H, H), lambda i: (0, 0, 0)),
        ],
        out_specs=pl.BlockSpec((nrel, BN, H), lambda i: (0, i, 0)),
        out_shape=jax.ShapeDtypeStruct((nrel, NPAD, H), jnp.float32),
    )(x, w)


def _xform2_body(nrel, p_ref, b_ref, w_ref, out_ref):
    h = jnp.maximum(p_ref[0] + p_ref[1] + b_ref[...], 0.0)
    for r in range(nrel):
        out_ref[r] = jnp.dot(h, w_ref[r], preferred_element_type=jnp.float32,
                             precision=lax.Precision.HIGHEST)


def _transform2(parts, bias, w):
    nrel = w.shape[0]
    grid = NPAD // BN
    return pl.pallas_call(
        functools.partial(_xform2_body, nrel),
        grid=(grid,),
        in_specs=[
            pl.BlockSpec((2, BN, H), lambda i: (0, i, 0)),
            pl.BlockSpec((1, H), lambda i: (0, 0)),
            pl.BlockSpec((nrel, H, H), lambda i: (0, 0, 0)),
        ],
        out_specs=pl.BlockSpec((nrel, BN, H), lambda i: (0, i, 0)),
        out_shape=jax.ShapeDtypeStruct((nrel, NPAD, H), jnp.float32),
    )(parts, bias.reshape(1, H), w)


def _combine_body(p_ref, b_ref, out_ref):
    out_ref[...] = p_ref[0] + p_ref[1] + b_ref[...]


def _combine(parts, bias):
    grid = NPAD // BN
    return pl.pallas_call(
        _combine_body,
        grid=(grid,),
        in_specs=[
            pl.BlockSpec((2, BN, H), lambda i: (0, i, 0)),
            pl.BlockSpec((1, H), lambda i: (0, 0)),
        ],
        out_specs=pl.BlockSpec((BN, H), lambda i: (i, 0)),
        out_shape=jax.ShapeDtypeStruct((NPAD, H), jnp.float32),
    )(parts, bias.reshape(1, H))


def _sc_edge_body(nchunk, hx_hbm, et_hbm, sr_hbm, ds_hbm, nm_hbm, out_hbm,
                  ev, sv, dv, nv, comb, combj, dvj, rows, acc, sem):
    c = lax.axis_index("c")
    s = lax.axis_index("s")
    wid = c * NS + s
    rows_per_tile = NPAD // NS

    @pl.loop(0, CHUNK)
    def _zero_loop(i):
        for g in range(H // L):
            rows[i, pl.ds(g * L, L)] = jnp.zeros((L,), jnp.float32)
    for k in range(rows_per_tile // CHUNK):
        pltpu.sync_copy(rows, acc.at[pl.ds(s * rows_per_tile + k * CHUNK,
                                           CHUNK)])
    plsc.subcore_barrier()

    @pl.loop(0, nchunk // SB)
    def _super_loop(sb):
        sl_sb = pl.ds(pl.multiple_of(wid * nchunk + sb * SB, 8), SB)
        pltpu.sync_copy(et_hbm.at[sl_sb], ev)
        pltpu.sync_copy(sr_hbm.at[sl_sb], sv)
        pltpu.sync_copy(ds_hbm.at[sl_sb], dv)
        pltpu.sync_copy(nm_hbm.at[sl_sb], nv)

        @pl.loop(0, SB)
        def _comb_loop(j):
            for i in range(CHUNK // L):
                sl = pl.ds(i * L, L)
                comb[j, sl] = ev[j, sl] * NPAD + sv[j, sl]

        @pl.loop(0, SB)
        def _edge_loop(j):
            jg = sb * SB + j

            @pl.when(jnp.logical_or(jg < HOLE_LO, jg >= HOLE_HI))
            def _work():
                for g in range(CHUNK // L):
                    sl = pl.ds(g * L, L)
                    combj[sl] = comb[j, sl]
                    dvj[sl] = dv[j, sl]
                pltpu.async_copy(hx_hbm.at[combj], rows, sem).wait()

                @pl.loop(0, CHUNK // L)
                def _scale_loop(ib):
                    nl = nv[j, pl.ds(ib * L, L)]
                    for k in range(L):
                        nsp = nl[k]
                        i = ib * L + k
                        for g in range(H // L):
                            sl = pl.ds(g * L, L)
                            rows[i, sl] = rows[i, sl] * nsp

                pltpu.sync_copy(rows, acc.at[dvj], add=True)

    plsc.subcore_barrier()
    pltpu.sync_copy(acc.at[pl.ds(s * rows_per_tile, rows_per_tile)],
                    out_hbm.at[c, pl.ds(s * rows_per_tile, rows_per_tile)])


def _sc_edge(hx_flat, et3, sr3, ds3, nm3, nchunk):
    mesh = plsc.VectorSubcoreMesh(core_axis_name="c", subcore_axis_name="s",
                                  num_cores=NC, num_subcores=NS)
    f = pl.kernel(
        functools.partial(_sc_edge_body, nchunk),
        out_type=jax.ShapeDtypeStruct((NC, NPAD, H), jnp.float32),
        mesh=mesh,
        scratch_types=[
            pltpu.VMEM((SB, CHUNK), jnp.int32),
            pltpu.VMEM((SB, CHUNK), jnp.int32),
            pltpu.VMEM((SB, CHUNK), jnp.int32),
            pltpu.VMEM((SB, CHUNK), jnp.float32),
            pltpu.VMEM((SB, CHUNK), jnp.int32),
            pltpu.VMEM((CHUNK,), jnp.int32),
            pltpu.VMEM((CHUNK,), jnp.int32),
            pltpu.VMEM((CHUNK, H), jnp.float32),
            pltpu.VMEM_SHARED((NPAD, H), jnp.float32),
            pltpu.SemaphoreType.DMA,
        ],
    )
    return f(hx_flat, et3, sr3, ds3, nm3)


def _prep_edges(edge_index, etype, norm):
    e = etype.shape[0]
    i32 = jnp.int32
    rows_pad = 4096
    real_rows = 80

    k = jnp.arange(real_rows)
    slab_row = jnp.where(k < HOLE_LO, k, k + (HOLE_HI - HOLE_LO))
    row_idx = (jnp.arange(NW)[:, None] * SLAB + slab_row[None, :]).reshape(-1)

    def shape2d(x, fill):
        padn = NW * real_rows * CHUNK - e
        full = jnp.concatenate([x, jnp.full((padn,), fill, x.dtype)])
        full = full.reshape(NW * real_rows, CHUNK)
        out = jnp.full((rows_pad, CHUNK), fill, x.dtype)
        return out.at[row_idx].set(full)

    src = shape2d(edge_index[0].astype(i32), 0)
    dst = shape2d(edge_index[1].astype(i32), NPAD - 1)
    et = shape2d(etype.astype(i32), 0)
    nm = shape2d(norm.reshape(-1), 0.0)
    return et, src, dst, nm, SLAB


def kernel(node_ids, edge_index1, etype1, norm1, edge_index2, etype2, norm2,
           emb, basis1, coeff1, bias1, basis2, coeff2, bias2):
    n = emb.shape[0]
    nrel = coeff1.shape[0]

    x = emb[node_ids]
    x = jnp.pad(x, ((0, NPAD - n), (0, 0)))

    w1 = _basis_weights(coeff1, basis1)
    w2 = _basis_weights(coeff2, basis2)

    hx1 = _transform1(x, w1)
    e1t, e1s, e1d, e1n, nchunk1 = _prep_edges(edge_index1, etype1, norm1)
    parts1 = _sc_edge(hx1.reshape(nrel * NPAD, H), e1t, e1s, e1d, e1n,
                      nchunk1)

    hx2 = _transform2(parts1, bias1, w2)
    e2t, e2s, e2d, e2n, nchunk2 = _prep_edges(edge_index2, etype2, norm2)
    parts2 = _sc_edge(hx2.reshape(nrel * NPAD, H), e2t, e2s, e2d, e2n,
                      nchunk2)

    out = _combine(parts2, bias2)
    return out[:n]
